# Initial kernel scaffold; baseline (speedup 1.0000x reference)
import jax
import jax.numpy as jnp
from jax.experimental import pallas as pl


def kernel(x, Wg, bg, W1, b1, W2, b2):
    raise NotImplementedError("write your pallas kernel here")


# speedup vs baseline: 1.6400x; 1.6400x over previous
import functools
import jax
import jax.numpy as jnp
from jax.experimental import pallas as pl
from jax.experimental.pallas import tpu as pltpu

_N = 2048
_D = 1024
_F = 2048
_E = 8
_FB = 512
_NFB = _F // _FB


def _moe_body(x_ref, wg_ref, bg_ref, w1_ref, b1_ref, w2_ref, b2_ref,
              out_ref, lb_ref, wtok_ref, xbf_ref):
    e = pl.program_id(0)
    f = pl.program_id(1)

    @pl.when((e == 0) & (f == 0))
    def _router():
        xs = x_ref[...]
        xbf_ref[...] = xs.astype(jnp.bfloat16)
        logits = jnp.dot(xs, wg_ref[...], preferred_element_type=jnp.float32)
        logits = logits + bg_ref[...]
        m = jnp.max(logits, axis=1, keepdims=True)
        ex = jnp.exp(logits - m)
        probs = ex / jnp.sum(ex, axis=1, keepdims=True)
        lane = jax.lax.broadcasted_iota(jnp.int32, (_N, _E), 1)
        m1 = jnp.max(probs, axis=1, keepdims=True)
        i1 = jnp.min(jnp.where(probs == m1, lane, _E), axis=1, keepdims=True)
        masked = jnp.where(lane == i1, -jnp.inf, probs)
        m2 = jnp.max(masked, axis=1, keepdims=True)
        i2 = jnp.min(jnp.where(masked == m2, lane, _E), axis=1, keepdims=True)
        denom = m1 + m2
        w1n = m1 / denom
        w2n = m2 / denom
        wtok_ref[...] = jnp.where(lane == i1, w1n,
                                  jnp.where(lane == i2, w2n, 0.0))
        colmean = jnp.mean(probs, axis=0, keepdims=True)
        mu = jnp.mean(colmean)
        lb_ref[...] = (jnp.sum((colmean - mu) ** 2) / (_E - 1)).reshape(1, 1)

    w1 = w1_ref[0].astype(jnp.bfloat16)
    h = jnp.dot(xbf_ref[...], w1, preferred_element_type=jnp.float32)
    h = jnp.maximum(h + b1_ref[0], 0.0)
    w2 = w2_ref[0].astype(jnp.bfloat16)
    y = jnp.dot(h.astype(jnp.bfloat16), w2, preferred_element_type=jnp.float32)

    lane = jax.lax.broadcasted_iota(jnp.int32, (1, _E), 1)
    we = jnp.sum(wtok_ref[...] * (lane == e), axis=1, keepdims=True)

    contrib = we * y

    @pl.when(f == 0)
    def _bias2():
        contrib_b = contrib + we * b2_ref[0]
        out_ref[...] = jnp.where(e == 0, 0.0, out_ref[...]) + contrib_b

    @pl.when(f != 0)
    def _acc():
        out_ref[...] += contrib


@functools.partial(jax.jit, static_argnames=("interpret",))
def kernel(x, Wg, bg, W1, b1, W2, b2, interpret=False):
    xs = x.reshape(_N, _D)
    out, lb = pl.pallas_call(
        _moe_body,
        grid=(_E, _NFB),
        in_specs=[
            pl.BlockSpec((_N, _D), lambda e, f: (0, 0)),
            pl.BlockSpec((_D, _E), lambda e, f: (0, 0)),
            pl.BlockSpec((1, _E), lambda e, f: (0, 0)),
            pl.BlockSpec((1, _D, _FB), lambda e, f: (e, 0, f)),
            pl.BlockSpec((1, 1, _FB), lambda e, f: (e, 0, f)),
            pl.BlockSpec((1, _FB, _D), lambda e, f: (e, f, 0)),
            pl.BlockSpec((1, 1, _D), lambda e, f: (e, 0, 0)),
        ],
        out_specs=[
            pl.BlockSpec((_N, _D), lambda e, f: (0, 0)),
            pl.BlockSpec((1, 1), lambda e, f: (0, 0)),
        ],
        out_shape=[
            jax.ShapeDtypeStruct((_N, _D), jnp.float32),
            jax.ShapeDtypeStruct((1, 1), jnp.float32),
        ],
        scratch_shapes=[
            pltpu.VMEM((_N, _E), jnp.float32),
            pltpu.VMEM((_N, _D), jnp.bfloat16),
        ],
        interpret=interpret,
    )(xs, Wg, bg.reshape(1, _E), W1, b1.reshape(_E, 1, _F), W2,
      b2.reshape(_E, 1, _D))
    return out.reshape(x.shape), lb[0, 0]
